# baseline (speedup 1.0000x reference)
import jax
import jax.numpy as jnp
from jax.experimental import pallas as pl
from jax.experimental.pallas import tpu as pltpu

XR, YR, ZR = 128, 192, 128
VOX = 2.0 / 192.0
SIG = 2.0 / 192.0
INV2S2 = 1.0 / (2.0 * SIG * SIG)
NV = 6890
VPAD = 6912
XBLK = 8
NBLK = XR // XBLK
KWIN = 1024


def _axis_weights(vmask, coord_vec, idx, n):
    base = jnp.floor(coord_vec / VOX + (0.5 * n - 0.5))
    center = (idx + (0.5 - 0.5 * n)) * VOX
    d = center - coord_vec
    w = jnp.exp(-(d * d) * INV2S2)
    mask = (idx >= base - 3.0) & (idx <= base + 3.0) & vmask
    return w * mask.astype(jnp.float32)


def _tables_kernel(vxr_ref, vy_ref, vz_ref, code_ref,
                   wxt_ref, wy_ref, b_ref):
    vmask_l = jax.lax.broadcasted_iota(jnp.int32, (1, VPAD), 1) < NV
    xi = jax.lax.broadcasted_iota(jnp.int32, (XR, 1), 0).astype(jnp.float32)
    wxt_ref[...] = _axis_weights(vmask_l, vxr_ref[...], xi, XR)
    vmask = jax.lax.broadcasted_iota(jnp.int32, (VPAD, 1), 0) < NV
    yi = jax.lax.broadcasted_iota(jnp.int32, (1, YR), 1).astype(jnp.float32)
    wy_ref[...] = _axis_weights(vmask, vy_ref[...], yi, YR)

    zi = jax.lax.broadcasted_iota(jnp.int32, (1, ZR), 1).astype(jnp.float32)
    wz = _axis_weights(vmask, vz_ref[...], zi, ZR)
    b_ref[:, 3 * ZR:] = wz.astype(jnp.bfloat16)

    li = jax.lax.broadcasted_iota(jnp.int32, (1, 3 * ZR), 1)
    zi3 = (li // 3).astype(jnp.float32)
    wz3 = _axis_weights(vmask, vz_ref[...], zi3, ZR)
    ci = li % 3
    csel = jnp.where(ci == 0, code_ref[:, 0:1],
                     jnp.where(ci == 1, code_ref[:, 1:2], code_ref[:, 2:3]))
    b_ref[:, :3 * ZR] = (wz3 * csel).astype(jnp.bfloat16)


def _accum_kernel(s_ref, full_ref, wx3_ref, wy_ref, b_ref, occ_ref,
                  osem_ref, ow_ref):
    erow = jax.lax.broadcasted_iota(jnp.int32, (ZR, 3 * ZR), 0)
    ecol = jax.lax.broadcasted_iota(jnp.int32, (ZR, 3 * ZR), 1)
    emat = (ecol // 3 == erow).astype(jnp.bfloat16)
    i = pl.program_id(0)
    dims = (((0,), (0,)), ((), ()))
    for x in range(XBLK):
        xg = i * XBLK + x
        st = s_ref[xg] * 8

        def _windowed():
            col = wx3_ref[0, pl.ds(st, KWIN), x:x + 1]
            m = (wy_ref[pl.ds(st, KWIN), :] * col).astype(jnp.bfloat16)
            return jax.lax.dot_general(
                m, b_ref[pl.ds(st, KWIN), :], dims,
                preferred_element_type=jnp.float32)

        def _full():
            col = wx3_ref[0, :, x:x + 1]
            m = (wy_ref[...] * col).astype(jnp.bfloat16)
            return jax.lax.dot_general(
                m, b_ref[...], dims,
                preferred_element_type=jnp.float32)

        acc = jax.lax.cond(full_ref[xg] == 0, _windowed, _full)
        gate = (occ_ref[x] > 1e-3).astype(jnp.bfloat16)
        gate3 = jax.lax.dot_general(
            gate, emat, (((1,), (0,)), ((), ())),
            preferred_element_type=jnp.float32)
        osem_ref[x] = acc[:, :3 * ZR] * gate3
        ow_ref[x] = acc[:, 3 * ZR:] * gate.astype(jnp.float32) + 1e-3


def kernel(smpl_vertices, occ_volume, smpl_vertex_code, smpl_face_indices):
    del smpl_face_indices

    base_x = jnp.floor(
        smpl_vertices[:, 0] / VOX + (0.5 * XR - 0.5)).astype(jnp.int32)
    order = jnp.argsort(base_x)
    verts = jnp.take(smpl_vertices, order, axis=0)
    code = jnp.take(smpl_vertex_code, order, axis=0)
    xs = jnp.arange(XR, dtype=jnp.int32)
    lo = jnp.sum((base_x[None, :] < (xs[:, None] - 3)), axis=1)
    hi = jnp.sum((base_x[None, :] <= (xs[:, None] + 3)), axis=1)
    start = jnp.minimum(lo, VPAD - KWIN)
    start8 = start // 8
    full = ((hi - start8 * 8) > KWIN).astype(jnp.int32)
    start8 = jnp.where(full == 1, 0, start8).astype(jnp.int32)

    pad = VPAD - NV
    verts = jnp.pad(verts, ((0, pad), (0, 0)))
    code = jnp.pad(code, ((0, pad), (0, 0)))
    vxr = verts[:, 0].reshape(1, VPAD)
    vy = verts[:, 1].reshape(VPAD, 1)
    vz = verts[:, 2].reshape(VPAD, 1)

    wxt, wy, bmat = pl.pallas_call(
        _tables_kernel,
        out_shape=[
            jax.ShapeDtypeStruct((XR, VPAD), jnp.float32),
            jax.ShapeDtypeStruct((VPAD, YR), jnp.float32),
            jax.ShapeDtypeStruct((VPAD, 4 * ZR), jnp.bfloat16),
        ],
    )(vxr, vy, vz, code)
    wx3 = wxt.reshape(NBLK, XBLK, VPAD).transpose(0, 2, 1)

    grid_spec = pltpu.PrefetchScalarGridSpec(
        num_scalar_prefetch=2,
        grid=(NBLK,),
        in_specs=[
            pl.BlockSpec((1, VPAD, XBLK), lambda i, s, f: (i, 0, 0)),
            pl.BlockSpec((VPAD, YR), lambda i, s, f: (0, 0)),
            pl.BlockSpec((VPAD, 4 * ZR), lambda i, s, f: (0, 0)),
            pl.BlockSpec((XBLK, YR, ZR), lambda i, s, f: (i, 0, 0)),
        ],
        out_specs=[
            pl.BlockSpec((XBLK, YR, 3 * ZR), lambda i, s, f: (i, 0, 0)),
            pl.BlockSpec((XBLK, YR, ZR), lambda i, s, f: (i, 0, 0)),
        ],
    )
    osem, ow = pl.pallas_call(
        _accum_kernel,
        grid_spec=grid_spec,
        out_shape=[
            jax.ShapeDtypeStruct((XR, YR, 3 * ZR), jnp.float32),
            jax.ShapeDtypeStruct((XR, YR, ZR), jnp.float32),
        ],
        compiler_params=pltpu.CompilerParams(
            dimension_semantics=("arbitrary",)),
    )(start8, full, wx3, wy, bmat, occ_volume)

    semantic_volume = osem.reshape(XR, YR, ZR, 3)
    weight_sum_volume = ow
    return semantic_volume, weight_sum_volume


# speedup vs baseline: 1.3266x; 1.3266x over previous
import jax
import jax.numpy as jnp
from jax.experimental import pallas as pl
from jax.experimental.pallas import tpu as pltpu

XR, YR, ZR = 128, 192, 128
VOX = 2.0 / 192.0
SIG = 2.0 / 192.0
INV2S2 = 1.0 / (2.0 * SIG * SIG)
NV = 6890
VPAD = 6912
XBLK = 8
NBLK = XR // XBLK
KBLK = 1536


def _axis_weights(vmask, coord_vec, idx, n):
    base = jnp.floor(coord_vec / VOX + (0.5 * n - 0.5))
    center = (idx + (0.5 - 0.5 * n)) * VOX
    d = center - coord_vec
    w = jnp.exp(-(d * d) * INV2S2)
    mask = (idx >= base - 3.0) & (idx <= base + 3.0) & vmask
    return w * mask.astype(jnp.float32)


def _tables_kernel(vx_ref, vy_ref, vz_ref, code_ref,
                   wxt_ref, wyt_ref, b_ref):
    vmask_l = jax.lax.broadcasted_iota(jnp.int32, (1, VPAD), 1) < NV
    xi = jax.lax.broadcasted_iota(jnp.int32, (XR, 1), 0).astype(jnp.float32)
    wxt_ref[...] = _axis_weights(vmask_l, vx_ref[...], xi, XR)
    yi = jax.lax.broadcasted_iota(jnp.int32, (YR, 1), 0).astype(jnp.float32)
    wyt_ref[...] = _axis_weights(vmask_l, vy_ref[...], yi, YR)

    vmask_s = jax.lax.broadcasted_iota(jnp.int32, (VPAD, 1), 0) < NV
    zi = jax.lax.broadcasted_iota(jnp.int32, (1, ZR), 1).astype(jnp.float32)
    wz = _axis_weights(vmask_s, vz_ref[...], zi, ZR)
    b_ref[:, 3 * ZR:] = wz.astype(jnp.bfloat16)

    li = jax.lax.broadcasted_iota(jnp.int32, (1, 3 * ZR), 1)
    zi3 = (li // 3).astype(jnp.float32)
    wz3 = _axis_weights(vmask_s, vz_ref[...], zi3, ZR)
    ci = li % 3
    csel = jnp.where(ci == 0, code_ref[:, 0:1],
                     jnp.where(ci == 1, code_ref[:, 1:2], code_ref[:, 2:3]))
    b_ref[:, :3 * ZR] = (wz3 * csel).astype(jnp.bfloat16)


def _gate_store(acc, occ_row, emat, osem_ref, ow_ref, x):
    gate = (occ_row > 1e-3).astype(jnp.bfloat16)
    gate3 = jax.lax.dot_general(
        gate, emat, (((1,), (0,)), ((), ())),
        preferred_element_type=jnp.float32)
    osem_ref[x] = acc[:, :3 * ZR] * gate3
    ow_ref[x] = acc[:, 3 * ZR:] * gate.astype(jnp.float32) + 1e-3


def _accum_kernel(s_ref, full_ref, wxt_ref, wyt_ref, b_ref, occ_ref,
                  osem_ref, ow_ref):
    erow = jax.lax.broadcasted_iota(jnp.int32, (ZR, 3 * ZR), 0)
    ecol = jax.lax.broadcasted_iota(jnp.int32, (ZR, 3 * ZR), 1)
    emat = (ecol // 3 == erow).astype(jnp.bfloat16)
    i = pl.program_id(0)
    st = s_ref[i] * 8
    dims = (((1,), (0,)), ((), ()))

    def _windowed(_):
        sh = jnp.where(st == 0, 0, VPAD - st)
        wyt_w = pltpu.roll(wyt_ref[...], sh, axis=1)[:, :KBLK]
        wxt_w = pltpu.roll(wxt_ref[...], sh, axis=1)[:, :KBLK]
        b_w = b_ref[pl.ds(st, KBLK), :]
        for x in range(XBLK):
            mt = (wyt_w * wxt_w[x:x + 1, :]).astype(jnp.bfloat16)
            acc = jax.lax.dot_general(
                mt, b_w, dims, preferred_element_type=jnp.float32)
            _gate_store(acc, occ_ref[x], emat, osem_ref, ow_ref, x)
        return 0

    def _full(_):
        wyt = wyt_ref[...]
        bmat = b_ref[...]
        for x in range(XBLK):
            mt = (wyt * wxt_ref[x:x + 1, :]).astype(jnp.bfloat16)
            acc = jax.lax.dot_general(
                mt, bmat, dims, preferred_element_type=jnp.float32)
            _gate_store(acc, occ_ref[x], emat, osem_ref, ow_ref, x)
        return 0

    jax.lax.cond(full_ref[i] == 0, _windowed, _full, 0)


def kernel(smpl_vertices, occ_volume, smpl_vertex_code, smpl_face_indices):
    del smpl_face_indices

    base_x = jnp.floor(
        smpl_vertices[:, 0] / VOX + (0.5 * XR - 0.5)).astype(jnp.int32)
    order = jnp.argsort(base_x)
    verts = jnp.take(smpl_vertices, order, axis=0)
    code = jnp.take(smpl_vertex_code, order, axis=0)
    xlo = jnp.arange(0, XR, XBLK, dtype=jnp.int32)
    xhi = xlo + (XBLK - 1)
    lo = jnp.sum((base_x[None, :] < (xlo[:, None] - 3)), axis=1)
    hi = jnp.sum((base_x[None, :] <= (xhi[:, None] + 3)), axis=1)
    start = jnp.minimum(lo, VPAD - KBLK)
    start8 = start // 8
    full = ((hi - start8 * 8) > KBLK).astype(jnp.int32)
    start8 = jnp.where(full == 1, 0, start8).astype(jnp.int32)

    pad = VPAD - NV
    verts = jnp.pad(verts, ((0, pad), (0, 0)))
    code = jnp.pad(code, ((0, pad), (0, 0)))
    vx = verts[:, 0].reshape(1, VPAD)
    vy = verts[:, 1].reshape(1, VPAD)
    vz = verts[:, 2].reshape(VPAD, 1)

    wxt, wyt, bmat = pl.pallas_call(
        _tables_kernel,
        out_shape=[
            jax.ShapeDtypeStruct((XR, VPAD), jnp.float32),
            jax.ShapeDtypeStruct((YR, VPAD), jnp.float32),
            jax.ShapeDtypeStruct((VPAD, 4 * ZR), jnp.bfloat16),
        ],
    )(vx, vy, vz, code)

    grid_spec = pltpu.PrefetchScalarGridSpec(
        num_scalar_prefetch=2,
        grid=(NBLK,),
        in_specs=[
            pl.BlockSpec((XBLK, VPAD), lambda i, s, f: (i, 0)),
            pl.BlockSpec((YR, VPAD), lambda i, s, f: (0, 0)),
            pl.BlockSpec((VPAD, 4 * ZR), lambda i, s, f: (0, 0)),
            pl.BlockSpec((XBLK, YR, ZR), lambda i, s, f: (i, 0, 0)),
        ],
        out_specs=[
            pl.BlockSpec((XBLK, YR, 3 * ZR), lambda i, s, f: (i, 0, 0)),
            pl.BlockSpec((XBLK, YR, ZR), lambda i, s, f: (i, 0, 0)),
        ],
    )
    osem, ow = pl.pallas_call(
        _accum_kernel,
        grid_spec=grid_spec,
        out_shape=[
            jax.ShapeDtypeStruct((XR, YR, 3 * ZR), jnp.float32),
            jax.ShapeDtypeStruct((XR, YR, ZR), jnp.float32),
        ],
        compiler_params=pltpu.CompilerParams(
            dimension_semantics=("arbitrary",)),
    )(start8, full, wxt, wyt, bmat, occ_volume)

    semantic_volume = osem.reshape(XR, YR, ZR, 3)
    weight_sum_volume = ow
    return semantic_volume, weight_sum_volume


# speedup vs baseline: 1.3645x; 1.0286x over previous
import jax
import jax.numpy as jnp
from jax.experimental import pallas as pl
from jax.experimental.pallas import tpu as pltpu

XR, YR, ZR = 128, 192, 128
VOX = 2.0 / 192.0
SIG = 2.0 / 192.0
INV2S2 = 1.0 / (2.0 * SIG * SIG)
NV = 6890
VPAD = 6912
XBLK = 8
NBLK = XR // XBLK
KBLK = 1536


def _axis_weights(vmask, coord_vec, idx, n):
    base = jnp.floor(coord_vec / VOX + (0.5 * n - 0.5))
    center = (idx + (0.5 - 0.5 * n)) * VOX
    d = center - coord_vec
    w = jnp.exp(-(d * d) * INV2S2)
    mask = (idx >= base - 3.0) & (idx <= base + 3.0) & vmask
    return w * mask.astype(jnp.float32)


def _tables_kernel(vx_ref, vy_ref, vz_ref, code_ref,
                   wxt_ref, wyt_ref, b_ref):
    vmask_l = jax.lax.broadcasted_iota(jnp.int32, (1, VPAD), 1) < NV
    xi = jax.lax.broadcasted_iota(jnp.int32, (XR, 1), 0).astype(jnp.float32)
    wxt_ref[...] = _axis_weights(vmask_l, vx_ref[...], xi, XR)
    yi = jax.lax.broadcasted_iota(jnp.int32, (YR, 1), 0).astype(jnp.float32)
    wyt_ref[...] = _axis_weights(vmask_l, vy_ref[...], yi, YR)

    vmask_s = jax.lax.broadcasted_iota(jnp.int32, (VPAD, 1), 0) < NV
    zi = jax.lax.broadcasted_iota(jnp.int32, (1, ZR), 1).astype(jnp.float32)
    wz = _axis_weights(vmask_s, vz_ref[...], zi, ZR)
    b_ref[:, 3 * ZR:] = wz.astype(jnp.bfloat16)

    erow = jax.lax.broadcasted_iota(jnp.int32, (ZR, 3 * ZR), 0)
    ecol = jax.lax.broadcasted_iota(jnp.int32, (ZR, 3 * ZR), 1)
    emat = (ecol // 3 == erow).astype(jnp.bfloat16)
    crow = jax.lax.broadcasted_iota(jnp.int32, (8, 3 * ZR), 0)
    ccol = jax.lax.broadcasted_iota(jnp.int32, (8, 3 * ZR), 1)
    cmat = (ccol % 3 == crow).astype(jnp.bfloat16)
    dims = (((1,), (0,)), ((), ()))
    wz3 = jax.lax.dot_general(
        wz.astype(jnp.bfloat16), emat, dims,
        preferred_element_type=jnp.float32)
    code8 = jnp.concatenate(
        [code_ref[...], jnp.zeros((VPAD, 5), jnp.float32)], axis=1)
    code3 = jax.lax.dot_general(
        code8.astype(jnp.bfloat16), cmat, dims,
        preferred_element_type=jnp.float32)
    b_ref[:, :3 * ZR] = (wz3 * code3).astype(jnp.bfloat16)


def _gate_store(acc, occ_row, emat, osem_ref, ow_ref, x):
    gate = (occ_row > 1e-3).astype(jnp.bfloat16)
    gate3 = jax.lax.dot_general(
        gate, emat, (((1,), (0,)), ((), ())),
        preferred_element_type=jnp.float32)
    osem_ref[x] = acc[:, :3 * ZR] * gate3
    ow_ref[x] = acc[:, 3 * ZR:] * gate.astype(jnp.float32) + 1e-3


def _accum_kernel(s_ref, full_ref, wxt_ref, wyt_ref, b_ref, occ_ref,
                  osem_ref, ow_ref):
    erow = jax.lax.broadcasted_iota(jnp.int32, (ZR, 3 * ZR), 0)
    ecol = jax.lax.broadcasted_iota(jnp.int32, (ZR, 3 * ZR), 1)
    emat = (ecol // 3 == erow).astype(jnp.bfloat16)
    i = pl.program_id(0)
    st = s_ref[i] * 8
    dims = (((1,), (0,)), ((), ()))

    def _windowed(_):
        sh = jnp.where(st == 0, 0, VPAD - st)
        wyt_w = pltpu.roll(wyt_ref[...], sh, axis=1)[:, :KBLK]
        wxt_w = pltpu.roll(wxt_ref[...], sh, axis=1)[:, :KBLK]
        b_w = b_ref[pl.ds(st, KBLK), :]
        for x in range(XBLK):
            mt = (wyt_w * wxt_w[x:x + 1, :]).astype(jnp.bfloat16)
            acc = jax.lax.dot_general(
                mt, b_w, dims, preferred_element_type=jnp.float32)
            _gate_store(acc, occ_ref[x], emat, osem_ref, ow_ref, x)
        return 0

    def _full(_):
        wyt = wyt_ref[...]
        bmat = b_ref[...]
        for x in range(XBLK):
            mt = (wyt * wxt_ref[x:x + 1, :]).astype(jnp.bfloat16)
            acc = jax.lax.dot_general(
                mt, bmat, dims, preferred_element_type=jnp.float32)
            _gate_store(acc, occ_ref[x], emat, osem_ref, ow_ref, x)
        return 0

    jax.lax.cond(full_ref[i] == 0, _windowed, _full, 0)


def kernel(smpl_vertices, occ_volume, smpl_vertex_code, smpl_face_indices):
    del smpl_face_indices

    base_x = jnp.floor(
        smpl_vertices[:, 0] / VOX + (0.5 * XR - 0.5)).astype(jnp.int32)
    order = jnp.argsort(base_x)
    verts = jnp.take(smpl_vertices, order, axis=0)
    code = jnp.take(smpl_vertex_code, order, axis=0)
    xlo = jnp.arange(0, XR, XBLK, dtype=jnp.int32)
    xhi = xlo + (XBLK - 1)
    lo = jnp.sum((base_x[None, :] < (xlo[:, None] - 3)), axis=1)
    hi = jnp.sum((base_x[None, :] <= (xhi[:, None] + 3)), axis=1)
    start = jnp.minimum(lo, VPAD - KBLK)
    start8 = start // 8
    full = ((hi - start8 * 8) > KBLK).astype(jnp.int32)
    start8 = jnp.where(full == 1, 0, start8).astype(jnp.int32)

    pad = VPAD - NV
    verts = jnp.pad(verts, ((0, pad), (0, 0)))
    code = jnp.pad(code, ((0, pad), (0, 0)))
    vx = verts[:, 0].reshape(1, VPAD)
    vy = verts[:, 1].reshape(1, VPAD)
    vz = verts[:, 2].reshape(VPAD, 1)

    wxt, wyt, bmat = pl.pallas_call(
        _tables_kernel,
        out_shape=[
            jax.ShapeDtypeStruct((XR, VPAD), jnp.float32),
            jax.ShapeDtypeStruct((YR, VPAD), jnp.float32),
            jax.ShapeDtypeStruct((VPAD, 4 * ZR), jnp.bfloat16),
        ],
    )(vx, vy, vz, code)

    grid_spec = pltpu.PrefetchScalarGridSpec(
        num_scalar_prefetch=2,
        grid=(NBLK,),
        in_specs=[
            pl.BlockSpec((XBLK, VPAD), lambda i, s, f: (i, 0)),
            pl.BlockSpec((YR, VPAD), lambda i, s, f: (0, 0)),
            pl.BlockSpec((VPAD, 4 * ZR), lambda i, s, f: (0, 0)),
            pl.BlockSpec((XBLK, YR, ZR), lambda i, s, f: (i, 0, 0)),
        ],
        out_specs=[
            pl.BlockSpec((XBLK, YR, 3 * ZR), lambda i, s, f: (i, 0, 0)),
            pl.BlockSpec((XBLK, YR, ZR), lambda i, s, f: (i, 0, 0)),
        ],
    )
    osem, ow = pl.pallas_call(
        _accum_kernel,
        grid_spec=grid_spec,
        out_shape=[
            jax.ShapeDtypeStruct((XR, YR, 3 * ZR), jnp.float32),
            jax.ShapeDtypeStruct((XR, YR, ZR), jnp.float32),
        ],
        compiler_params=pltpu.CompilerParams(
            dimension_semantics=("arbitrary",)),
    )(start8, full, wxt, wyt, bmat, occ_volume)

    semantic_volume = osem.reshape(XR, YR, ZR, 3)
    weight_sum_volume = ow
    return semantic_volume, weight_sum_volume


# speedup vs baseline: 1.4443x; 1.0585x over previous
import jax
import jax.numpy as jnp
from jax.experimental import pallas as pl
from jax.experimental.pallas import tpu as pltpu

XR, YR, ZR = 128, 192, 128
VOX = 2.0 / 192.0
SIG = 2.0 / 192.0
INV2S2 = 1.0 / (2.0 * SIG * SIG)
NV = 6890
VPAD = 6912
XBLK = 8
NBLK = XR // XBLK
KBLK = 1024


def _axis_weights(vmask, coord_vec, idx, n):
    base = jnp.floor(coord_vec / VOX + (0.5 * n - 0.5))
    center = (idx + (0.5 - 0.5 * n)) * VOX
    d = center - coord_vec
    w = jnp.exp(-(d * d) * INV2S2)
    mask = (idx >= base - 3.0) & (idx <= base + 3.0) & vmask
    return w * mask.astype(jnp.float32)


def _tables_kernel(vx_ref, vy_ref, vz_ref, code_ref,
                   wxt_ref, wyt_ref, b_ref):
    vmask_l = jax.lax.broadcasted_iota(jnp.int32, (1, VPAD), 1) < NV
    xi = jax.lax.broadcasted_iota(jnp.int32, (XR, 1), 0).astype(jnp.float32)
    wxt_ref[...] = _axis_weights(vmask_l, vx_ref[...], xi, XR)
    yi = jax.lax.broadcasted_iota(jnp.int32, (YR, 1), 0).astype(jnp.float32)
    wyt_ref[...] = _axis_weights(vmask_l, vy_ref[...], yi, YR)

    vmask_s = jax.lax.broadcasted_iota(jnp.int32, (VPAD, 1), 0) < NV
    zi = jax.lax.broadcasted_iota(jnp.int32, (1, ZR), 1).astype(jnp.float32)
    wz = _axis_weights(vmask_s, vz_ref[...], zi, ZR)
    b_ref[:, 3 * ZR:] = wz.astype(jnp.bfloat16)

    erow = jax.lax.broadcasted_iota(jnp.int32, (ZR, 3 * ZR), 0)
    ecol = jax.lax.broadcasted_iota(jnp.int32, (ZR, 3 * ZR), 1)
    emat = (ecol // 3 == erow).astype(jnp.bfloat16)
    crow = jax.lax.broadcasted_iota(jnp.int32, (8, 3 * ZR), 0)
    ccol = jax.lax.broadcasted_iota(jnp.int32, (8, 3 * ZR), 1)
    cmat = (ccol % 3 == crow).astype(jnp.bfloat16)
    dims = (((1,), (0,)), ((), ()))
    wz3 = jax.lax.dot_general(
        wz.astype(jnp.bfloat16), emat, dims,
        preferred_element_type=jnp.float32)
    code8 = jnp.concatenate(
        [code_ref[...], jnp.zeros((VPAD, 5), jnp.float32)], axis=1)
    code3 = jax.lax.dot_general(
        code8.astype(jnp.bfloat16), cmat, dims,
        preferred_element_type=jnp.float32)
    b_ref[:, :3 * ZR] = (wz3 * code3).astype(jnp.bfloat16)


def _gate_store(acc, occ_row, emat, osem_ref, ow_ref, x):
    gate = (occ_row > 1e-3).astype(jnp.bfloat16)
    gate3 = jax.lax.dot_general(
        gate, emat, (((1,), (0,)), ((), ())),
        preferred_element_type=jnp.float32)
    osem_ref[x] = acc[:, :3 * ZR] * gate3
    ow_ref[x] = acc[:, 3 * ZR:] * gate.astype(jnp.float32) + 1e-3


def _accum_kernel(s_ref, full_ref, wxt_ref, wyt_ref, b_ref, occ_ref,
                  osem_ref, ow_ref):
    erow = jax.lax.broadcasted_iota(jnp.int32, (ZR, 3 * ZR), 0)
    ecol = jax.lax.broadcasted_iota(jnp.int32, (ZR, 3 * ZR), 1)
    emat = (ecol // 3 == erow).astype(jnp.bfloat16)
    i = pl.program_id(0)
    st = s_ref[i] * 8
    dims = (((1,), (0,)), ((), ()))

    def _windowed(_):
        sh = jnp.where(st == 0, 0, VPAD - st)
        wyt_w = pltpu.roll(wyt_ref[...], sh, axis=1)[:, :KBLK]
        wxt_w = pltpu.roll(wxt_ref[...], sh, axis=1)[:, :KBLK]
        b_w = b_ref[pl.ds(st, KBLK), :]
        for x in range(XBLK):
            mt = (wyt_w * wxt_w[x:x + 1, :]).astype(jnp.bfloat16)
            acc = jax.lax.dot_general(
                mt, b_w, dims, preferred_element_type=jnp.float32)
            _gate_store(acc, occ_ref[x], emat, osem_ref, ow_ref, x)
        return 0

    def _full(_):
        wyt = wyt_ref[...]
        bmat = b_ref[...]
        for x in range(XBLK):
            mt = (wyt * wxt_ref[x:x + 1, :]).astype(jnp.bfloat16)
            acc = jax.lax.dot_general(
                mt, bmat, dims, preferred_element_type=jnp.float32)
            _gate_store(acc, occ_ref[x], emat, osem_ref, ow_ref, x)
        return 0

    jax.lax.cond(full_ref[i] == 0, _windowed, _full, 0)


def kernel(smpl_vertices, occ_volume, smpl_vertex_code, smpl_face_indices):
    del smpl_face_indices

    base_x = jnp.floor(
        smpl_vertices[:, 0] / VOX + (0.5 * XR - 0.5)).astype(jnp.int32)
    order = jnp.argsort(base_x)
    verts = jnp.take(smpl_vertices, order, axis=0)
    code = jnp.take(smpl_vertex_code, order, axis=0)
    xlo = jnp.arange(0, XR, XBLK, dtype=jnp.int32)
    xhi = xlo + (XBLK - 1)
    lo = jnp.sum((base_x[None, :] < (xlo[:, None] - 3)), axis=1)
    hi = jnp.sum((base_x[None, :] <= (xhi[:, None] + 3)), axis=1)
    start = jnp.minimum(lo, VPAD - KBLK)
    start8 = start // 8
    full = ((hi - start8 * 8) > KBLK).astype(jnp.int32)
    start8 = jnp.where(full == 1, 0, start8).astype(jnp.int32)

    pad = VPAD - NV
    verts = jnp.pad(verts, ((0, pad), (0, 0)))
    code = jnp.pad(code, ((0, pad), (0, 0)))
    vx = verts[:, 0].reshape(1, VPAD)
    vy = verts[:, 1].reshape(1, VPAD)
    vz = verts[:, 2].reshape(VPAD, 1)

    wxt, wyt, bmat = pl.pallas_call(
        _tables_kernel,
        out_shape=[
            jax.ShapeDtypeStruct((XR, VPAD), jnp.float32),
            jax.ShapeDtypeStruct((YR, VPAD), jnp.float32),
            jax.ShapeDtypeStruct((VPAD, 4 * ZR), jnp.bfloat16),
        ],
    )(vx, vy, vz, code)

    grid_spec = pltpu.PrefetchScalarGridSpec(
        num_scalar_prefetch=2,
        grid=(NBLK,),
        in_specs=[
            pl.BlockSpec((XBLK, VPAD), lambda i, s, f: (i, 0)),
            pl.BlockSpec((YR, VPAD), lambda i, s, f: (0, 0)),
            pl.BlockSpec((VPAD, 4 * ZR), lambda i, s, f: (0, 0)),
            pl.BlockSpec((XBLK, YR, ZR), lambda i, s, f: (i, 0, 0)),
        ],
        out_specs=[
            pl.BlockSpec((XBLK, YR, 3 * ZR), lambda i, s, f: (i, 0, 0)),
            pl.BlockSpec((XBLK, YR, ZR), lambda i, s, f: (i, 0, 0)),
        ],
    )
    osem, ow = pl.pallas_call(
        _accum_kernel,
        grid_spec=grid_spec,
        out_shape=[
            jax.ShapeDtypeStruct((XR, YR, 3 * ZR), jnp.float32),
            jax.ShapeDtypeStruct((XR, YR, ZR), jnp.float32),
        ],
        compiler_params=pltpu.CompilerParams(
            dimension_semantics=("arbitrary",)),
    )(start8, full, wxt, wyt, bmat, occ_volume)

    semantic_volume = osem.reshape(XR, YR, ZR, 3)
    weight_sum_volume = ow
    return semantic_volume, weight_sum_volume
